# baseline (speedup 1.0000x reference)
import functools

import jax
import jax.numpy as jnp
from jax import lax
from jax.experimental import pallas as pl
from jax.experimental.pallas import tpu as pltpu
from jax.experimental.pallas import tpu_sc as plsc

BATCH = 4
SEQLEN = 2048
EMBED = 64
NUM_CORES = 2
NUM_SUBCORES = 16
NW = NUM_CORES * NUM_SUBCORES
TOTAL = BATCH * SEQLEN
CHUNK = TOTAL // NW
GCH = 128
NG = CHUNK // GCH
LANES = 16
NGRP = CHUNK // LANES
VPAIR = 500000


@functools.partial(
    pl.kernel,
    out_type=jax.ShapeDtypeStruct((EMBED, TOTAL), jnp.float32),
    mesh=plsc.VectorSubcoreMesh(core_axis_name="c", subcore_axis_name="s"),
    scratch_types=[
        pltpu.VMEM((CHUNK,), jnp.int32),
        pltpu.VMEM((NG, GCH), jnp.int32),
        pltpu.VMEM((CHUNK, 2 * EMBED), jnp.float32),
        pltpu.VMEM((EMBED, CHUNK), jnp.float32),
        pltpu.VMEM((EMBED, CHUNK), jnp.float32),
        pltpu.SemaphoreType.DMA,
        pltpu.SemaphoreType.DMA,
    ],
    compiler_params=pltpu.CompilerParams(needs_layout_passes=False),
)
def _embed_lookup(x_hbm, tok2_hbm, posT_hbm, outT_hbm,
                  idx_v, pr_v, prow_v, posT_v, outT_v, gsem, psem):
    wid = lax.axis_index("s") * NUM_CORES + lax.axis_index("c")
    base = wid * CHUNK
    pos_base = lax.rem(base, SEQLEN)

    pltpu.sync_copy(x_hbm.at[pl.ds(base, CHUNK)], idx_v)
    pos_cp = pltpu.async_copy(
        posT_hbm.at[:, pl.ds(pos_base, CHUNK)], posT_v, psem
    )

    def mk_pairs(m, carry):
        v = idx_v[pl.ds(m * LANES, LANES)]
        k = m // (GCH // LANES)
        off = lax.rem(m, GCH // LANES) * LANES
        pr_v[k, pl.ds(off, LANES)] = lax.shift_right_logical(v, 1)
        return carry

    lax.fori_loop(0, CHUNK // LANES, mk_pairs, 0, unroll=4)

    gathers = [
        pltpu.async_copy(
            tok2_hbm.at[pr_v.at[k]], prow_v.at[pl.ds(k * GCH, GCH)], gsem
        )
        for k in range(NG)
    ]
    pos_cp.wait()
    for cp in gathers:
        cp.wait()

    iota = lax.iota(jnp.int32, LANES)

    def group(g, carry):
        rows = iota + g * LANES
        halfs = (idx_v[pl.ds(g * LANES, LANES)] & 1) * EMBED

        def comp(c, carry2):
            vals = plsc.load_gather(prow_v, [rows, halfs + c])
            sl = pl.ds(g * LANES, LANES)
            outT_v[c, sl] = vals + posT_v[c, sl]
            return carry2

        lax.fori_loop(0, EMBED, comp, 0, unroll=8)
        return carry

    lax.fori_loop(0, NGRP, group, 0)

    pltpu.sync_copy(outT_v, outT_hbm.at[:, pl.ds(base, CHUNK)])


def kernel(x, token_table, pos_table):
    tok2 = token_table.reshape(VPAIR, 2 * EMBED)
    outT = _embed_lookup(x.reshape(TOTAL), tok2, pos_table.T)
    return outT.T.reshape(BATCH, SEQLEN, EMBED).astype(jnp.bfloat16)


# speedup vs baseline: 1.6305x; 1.6305x over previous
import functools

import jax
import jax.numpy as jnp
from jax import lax
from jax.experimental import pallas as pl
from jax.experimental.pallas import tpu as pltpu
from jax.experimental.pallas import tpu_sc as plsc

BATCH = 4
SEQLEN = 2048
EMBED = 64
NUM_CORES = 2
NUM_SUBCORES = 16
NW = NUM_CORES * NUM_SUBCORES
TOTAL = BATCH * SEQLEN
CHUNK = TOTAL // NW
GCH = 64
NWAVE = CHUNK // GCH
LANES = 16
SUBS = 8


@functools.partial(
    pl.kernel,
    out_type=jax.ShapeDtypeStruct((EMBED, TOTAL), jnp.float32),
    mesh=plsc.VectorSubcoreMesh(core_axis_name="c", subcore_axis_name="s"),
    scratch_types=[
        pltpu.VMEM((CHUNK + LANES,), jnp.int32),
        pltpu.VMEM((GCH, SUBS, EMBED), jnp.float32),
        pltpu.VMEM((EMBED, CHUNK), jnp.float32),
        pltpu.VMEM((EMBED, CHUNK), jnp.float32),
        pltpu.SemaphoreType.DMA,
        pltpu.SemaphoreType.DMA,
    ],
    compiler_params=pltpu.CompilerParams(needs_layout_passes=False),
)
def _embed_lookup(x_hbm, tab_hbm, posT_hbm, outT_hbm,
                  idx_v, prow_v, posT_v, outT_v, gsem, psem):
    wid = lax.axis_index("s") * NUM_CORES + lax.axis_index("c")
    base = wid * CHUNK
    pos_base = lax.rem(base, SEQLEN)

    pltpu.sync_copy(x_hbm.at[pl.ds(base, CHUNK)], idx_v.at[pl.ds(0, CHUNK)])
    pos_cp = pltpu.async_copy(
        posT_hbm.at[:, pl.ds(pos_base, CHUNK)], posT_v, psem
    )
    pos_cp.wait()
    iota = lax.iota(jnp.int32, LANES)

    for w in range(NWAVE):
        def fire(j, carry, w=w):
            tok = idx_v[pl.ds(w * GCH + j, LANES)][0]
            t8 = lax.shift_right_logical(tok, 3) * SUBS
            pltpu.async_copy(
                tab_hbm.at[pl.ds(t8, SUBS), :], prow_v.at[j], gsem
            )
            return carry

        lax.fori_loop(0, GCH, fire, 0)

        def drain(j, carry):
            pltpu.make_async_copy(
                tab_hbm.at[pl.ds(0, SUBS), :], prow_v.at[j], gsem
            ).wait()
            return carry

        lax.fori_loop(0, GCH, drain, 0)

        def group(g, carry, w=w):
            tbase = w * GCH + g * LANES
            rows = iota + g * LANES
            subs = idx_v[pl.ds(tbase, LANES)] & (SUBS - 1)

            def comp(c, carry2):
                vals = plsc.load_gather(prow_v, [rows, subs, iota * 0 + c])
                sl = pl.ds(tbase, LANES)
                outT_v[c, sl] = vals + posT_v[c, sl]
                return carry2

            lax.fori_loop(0, EMBED, comp, 0, unroll=8)
            return carry

        lax.fori_loop(0, GCH // LANES, group, 0)

    pltpu.sync_copy(outT_v, outT_hbm.at[:, pl.ds(base, CHUNK)])


def kernel(x, token_table, pos_table):
    outT = _embed_lookup(x.reshape(TOTAL), token_table, pos_table.T)
    return outT.T.reshape(BATCH, SEQLEN, EMBED).astype(jnp.bfloat16)


# speedup vs baseline: 2.3919x; 1.4669x over previous
import functools

import jax
import jax.numpy as jnp
from jax import lax
from jax.experimental import pallas as pl
from jax.experimental.pallas import tpu as pltpu
from jax.experimental.pallas import tpu_sc as plsc

BATCH = 4
SEQLEN = 2048
EMBED = 64
NUM_CORES = 2
NUM_SUBCORES = 16
NW = NUM_CORES * NUM_SUBCORES
TOTAL = BATCH * SEQLEN
CHUNK = TOTAL // NW
GCH = 64
NWAVE = CHUNK // GCH
LANES = 16
SUBS = 8


@functools.partial(
    pl.kernel,
    out_type=jax.ShapeDtypeStruct((EMBED, TOTAL), jnp.float32),
    mesh=plsc.VectorSubcoreMesh(core_axis_name="c", subcore_axis_name="s"),
    scratch_types=[
        pltpu.VMEM((CHUNK + LANES,), jnp.int32),
        pltpu.VMEM((GCH, SUBS, EMBED), jnp.float32),
        pltpu.VMEM((EMBED, CHUNK), jnp.float32),
        pltpu.VMEM((EMBED, CHUNK), jnp.float32),
        pltpu.SemaphoreType.DMA,
        pltpu.SemaphoreType.DMA,
    ],
    compiler_params=pltpu.CompilerParams(needs_layout_passes=False),
)
def _embed_lookup(x_hbm, tab_hbm, posT_hbm, outT_hbm,
                  idx_v, prow_v, posT_v, outT_v, gsem, psem):
    wid = lax.axis_index("s") * NUM_CORES + lax.axis_index("c")
    base = wid * CHUNK
    pos_base = lax.rem(base, SEQLEN)

    pltpu.sync_copy(x_hbm.at[pl.ds(base, CHUNK)], idx_v.at[pl.ds(0, CHUNK)])
    pos_cp = pltpu.async_copy(
        posT_hbm.at[:, pl.ds(pos_base, CHUNK)], posT_v, psem
    )
    pos_cp.wait()
    iota = lax.iota(jnp.int32, LANES)

    for w in range(NWAVE):
        def fire(j, carry, w=w):
            tok = idx_v[pl.ds(w * GCH + j, LANES)][0]
            g = lax.shift_right_logical(tok, 3)
            pltpu.async_copy(tab_hbm.at[g], prow_v.at[j], gsem)
            return carry

        lax.fori_loop(0, GCH, fire, 0)

        def drain(j, carry):
            pltpu.make_async_copy(
                tab_hbm.at[0], prow_v.at[j], gsem
            ).wait()
            return carry

        lax.fori_loop(0, GCH, drain, 0)

        def group(g, carry, w=w):
            tbase = w * GCH + g * LANES
            rows = iota + g * LANES
            subs = idx_v[pl.ds(tbase, LANES)] & (SUBS - 1)

            def comp(c, carry2):
                vals = plsc.load_gather(prow_v, [rows, subs, iota * 0 + c])
                sl = pl.ds(tbase, LANES)
                outT_v[c, sl] = vals + posT_v[c, sl]
                return carry2

            lax.fori_loop(0, EMBED, comp, 0, unroll=8)
            return carry

        lax.fori_loop(0, GCH // LANES, group, 0)

    pltpu.sync_copy(outT_v, outT_hbm.at[:, pl.ds(base, CHUNK)])


def kernel(x, token_table, pos_table):
    tok3 = token_table.reshape(1000000 // SUBS, SUBS, EMBED)
    outT = _embed_lookup(x.reshape(TOTAL), tok3, pos_table.T)
    return outT.T.reshape(BATCH, SEQLEN, EMBED).astype(jnp.bfloat16)


# speedup vs baseline: 2.4383x; 1.0194x over previous
import functools

import jax
import jax.numpy as jnp
from jax import lax
from jax.experimental import pallas as pl
from jax.experimental.pallas import tpu as pltpu
from jax.experimental.pallas import tpu_sc as plsc

BATCH = 4
SEQLEN = 2048
EMBED = 64
NUM_CORES = 2
NUM_SUBCORES = 16
NW = NUM_CORES * NUM_SUBCORES
TOTAL = BATCH * SEQLEN
CHUNK = TOTAL // NW
WAVE = 16
NWAVE = CHUNK // WAVE
LANES = 16
SUBS = 8


@functools.partial(
    pl.kernel,
    out_type=jax.ShapeDtypeStruct((TOTAL, EMBED), jnp.float32),
    mesh=plsc.VectorSubcoreMesh(core_axis_name="c", subcore_axis_name="s"),
    scratch_types=[
        pltpu.VMEM((CHUNK + LANES,), jnp.int32),
        pltpu.VMEM((2, WAVE, SUBS, EMBED), jnp.float32),
        pltpu.VMEM((CHUNK, EMBED), jnp.float32),
        pltpu.VMEM((CHUNK, EMBED), jnp.float32),
        pltpu.SemaphoreType.DMA,
        pltpu.SemaphoreType.DMA,
        pltpu.SemaphoreType.DMA,
    ],
    compiler_params=pltpu.CompilerParams(needs_layout_passes=False),
)
def _embed_lookup(x_hbm, tab_hbm, pos_hbm, out_hbm,
                  idx_v, prow_v, pos_v, out_v, sem_a, sem_b, psem):
    wid = lax.axis_index("s") * NUM_CORES + lax.axis_index("c")
    base = wid * CHUNK
    pos_base = lax.rem(base, SEQLEN)

    pltpu.sync_copy(x_hbm.at[pl.ds(base, CHUNK)], idx_v.at[pl.ds(0, CHUNK)])
    pos_cp = pltpu.async_copy(pos_hbm.at[pl.ds(pos_base, CHUNK)], pos_v, psem)
    sems = (sem_a, sem_b)

    def fire_wave(w):
        b = w % 2

        def fire(j, carry):
            tok = idx_v[pl.ds(w * WAVE + j, LANES)][0]
            g = lax.shift_right_logical(tok, 3)
            pltpu.async_copy(tab_hbm.at[g], prow_v.at[b, j], sems[b])
            return carry

        lax.fori_loop(0, WAVE, fire, 0)

    def drain_wave(w):
        b = w % 2

        def drain(j, carry):
            pltpu.make_async_copy(
                tab_hbm.at[0], prow_v.at[b, j], sems[b]
            ).wait()
            return carry

        lax.fori_loop(0, WAVE, drain, 0)

    def compute_wave(w):
        b = w % 2

        def comp(j, carry):
            t = w * WAVE + j
            tok = idx_v[pl.ds(t, LANES)][0]
            s = tok & (SUBS - 1)
            for c in range(EMBED // LANES):
                sl = pl.ds(c * LANES, LANES)
                out_v[t, sl] = prow_v[b, j, s, sl] + pos_v[t, sl]
            return carry

        lax.fori_loop(0, WAVE, comp, 0)

    fire_wave(0)
    pos_cp.wait()
    for w in range(NWAVE):
        if w + 1 < NWAVE:
            fire_wave(w + 1)
        drain_wave(w)
        compute_wave(w)

    pltpu.sync_copy(out_v, out_hbm.at[pl.ds(base, CHUNK)])


def kernel(x, token_table, pos_table):
    tok3 = token_table.reshape(1000000 // SUBS, SUBS, EMBED)
    out = _embed_lookup(x.reshape(TOTAL), tok3, pos_table)
    return out.reshape(BATCH, SEQLEN, EMBED).astype(jnp.bfloat16)
